# baseline (speedup 1.0000x reference)
import functools

import jax
import jax.numpy as jnp
from jax import lax
from jax.experimental import pallas as pl
from jax.experimental.pallas import tpu as pltpu
from jax.experimental.pallas import tpu_sc as plsc

C = 128
N = 10000
K = 32
NK = N * K
NC, NS = 2, 16
NW = NC * NS
IDX_ROW = 128
ROWS = 2560
R_PER_W = ROWS // NW

NB = 200
F = NB * K


def _sc_gather(table, idx2d):
    mesh = plsc.VectorSubcoreMesh(core_axis_name="c", subcore_axis_name="s")

    @functools.partial(
        pl.kernel,
        mesh=mesh,
        out_type=jax.ShapeDtypeStruct((ROWS * IDX_ROW, C), jnp.float32),
        scratch_types=[
            pltpu.VMEM((IDX_ROW,), jnp.int32),
            pltpu.VMEM((IDX_ROW, C), jnp.float32),
            pltpu.SemaphoreType.DMA,
        ],
    )
    def k(table_hbm, idx_hbm, out_hbm, idx_v, rows_v, sem):
        wid = lax.axis_index("s") * NC + lax.axis_index("c")

        @pl.loop(0, R_PER_W)
        def _(i):
            r = wid * R_PER_W + i
            pltpu.sync_copy(idx_hbm.at[r], idx_v)
            pltpu.async_copy(table_hbm.at[idx_v], rows_v, sem).wait()
            pltpu.sync_copy(rows_v, out_hbm.at[pl.ds(r * IDX_ROW, IDX_ROW)])

    return k(table, idx2d)


def _tc_body(g_ref, xt_ref, o_ref):
    g = g_ref[...]
    xt = xt_ref[...]
    ctr = jnp.broadcast_to(xt[:, None, :], (NB, K, C)).reshape(F, C)
    both = jnp.concatenate([g - ctr, ctr], axis=1)
    o_ref[...] = both.T


def _tc_transform(g, x_t):
    return pl.pallas_call(
        _tc_body,
        grid=(N // NB,),
        in_specs=[
            pl.BlockSpec((F, C), lambda i: (i, 0)),
            pl.BlockSpec((NB, C), lambda i: (i, 0)),
        ],
        out_specs=pl.BlockSpec((2 * C, F), lambda i: (0, i)),
        out_shape=jax.ShapeDtypeStruct((2 * C, NK), jnp.float32),
    )(g, x_t)


def kernel(x, idx):
    b, c, n = x.shape
    k = idx.shape[-1]
    x_t = x[0].T
    idx_flat = idx.reshape(-1).astype(jnp.int32)
    idx_pad = jnp.zeros((ROWS * IDX_ROW,), jnp.int32).at[:NK].set(idx_flat)
    g = _sc_gather(x_t, idx_pad.reshape(ROWS, IDX_ROW))
    out2 = _tc_transform(g, x_t)
    return out2.reshape(1, 2 * c, n, k)
